# baseline (speedup 1.0000x reference)
import functools

import jax
import jax.numpy as jnp
from jax import lax
from jax.experimental import pallas as pl
from jax.experimental.pallas import tpu as pltpu
from jax.experimental.pallas import tpu_sc as plsc

D = 64
B = 16384
L = 16
NC = 2
NS = 16
NW = NC * NS
PER_W = B // NW
CHUNK = 64
NG = CHUNK // L
N_CHUNKS = PER_W // CHUNK
NUM_ENT = 1000000
NUM_REL = 1000


def _make_sc_kernel():
    mesh = plsc.VectorSubcoreMesh(core_axis_name="c", subcore_axis_name="s")

    @functools.partial(
        pl.kernel,
        out_type=jax.ShapeDtypeStruct((NW, L), jnp.float32),
        mesh=mesh,
        compiler_params=pltpu.CompilerParams(needs_layout_passes=False),
        scratch_types=[
            pltpu.VMEM((CHUNK,), jnp.int32),
            pltpu.VMEM((CHUNK,), jnp.int32),
            pltpu.VMEM((CHUNK,), jnp.int32),
            pltpu.VMEM((CHUNK,), jnp.int32),
            pltpu.VMEM((CHUNK, 2 * D), jnp.float32),
            pltpu.VMEM((CHUNK, 2 * D), jnp.float32),
            pltpu.VMEM((L, 8, D), jnp.float32),
            pltpu.VMEM((L, 8, D), jnp.float32),
            pltpu.VMEM((L, 8, D), jnp.float32),
            pltpu.VMEM((L, 8, D), jnp.float32),
            pltpu.VMEM((L,), jnp.float32),
            pltpu.SemaphoreType.DMA,
        ],
    )
    def sc_kernel(heads, rels, tails, ere3, eim3, rre, rim, out,
                  hidx, ridx, tidx, rdr, rbre, rbim,
                  bhre, bhim, btre, btim, accv, sem):
        wid = lax.axis_index("s") * NC + lax.axis_index("c")
        base = wid * PER_W

        def chunk_body(ck, accs):
            off = base + ck * CHUNK
            pltpu.sync_copy(heads.at[pl.ds(off, CHUNK)], hidx)
            pltpu.sync_copy(rels.at[pl.ds(off, CHUNK)], ridx)
            pltpu.sync_copy(tails.at[pl.ds(off, CHUNK)], tidx)
            for g in range(NG):
                sl = pl.ds(g * L, L)
                rdr[sl] = lax.shift_right_logical(ridx[sl], 1)
            cr1 = pltpu.async_copy(rre.at[rdr], rbre, sem)
            cr2 = pltpu.async_copy(rim.at[rdr], rbim, sem)
            cr1.wait()
            cr2.wait()

            def group_body(g, a):
                sl = pl.ds(g * L, L)
                hv = hidx[sl]
                tv = tidx[sl]
                rv = ridx[sl]
                hblk = lax.shift_right_logical(hv, 3)
                tblk = lax.shift_right_logical(tv, 3)
                hsub = lax.bitwise_and(hv, 7)
                tsub = lax.bitwise_and(tv, 7)
                rhalf = lax.bitwise_and(rv, 1) * D

                copies = []
                for l in range(L):
                    copies.append(pltpu.async_copy(
                        ere3.at[hblk[l]], bhre.at[l], sem))
                    copies.append(pltpu.async_copy(
                        eim3.at[hblk[l]], bhim.at[l], sem))
                    copies.append(pltpu.async_copy(
                        ere3.at[tblk[l]], btre.at[l], sem))
                    copies.append(pltpu.async_copy(
                        eim3.at[tblk[l]], btim.at[l], sem))

                del copies
                for r in (bhre, bhim, btre, btim):
                    pltpu.make_async_copy(
                        ere3.at[pl.ds(0, L)], r, sem).wait()

                new = list(a)
                for l in range(L):
                    t = g * L + l
                    hs = hsub[l]
                    ts = tsub[l]
                    rh = rhalf[l]
                    for j in range(D // L):
                        dsl = pl.ds(j * L, L)
                        vhr = bhre[l, hs, dsl]
                        vhi = bhim[l, hs, dsl]
                        vtr = btre[l, ts, dsl]
                        vti = btim[l, ts, dsl]
                        rsl = pl.ds(rh + j * L, L)
                        vrr = rbre[t, rsl]
                        vri = rbim[t, rsl]
                        new[j] = (new[j] + vrr * (vhr * vtr + vhi * vti)
                                  + vri * (vhr * vti - vhi * vtr))
                return tuple(new)

            return lax.fori_loop(0, NG, group_body, accs)

        accs = lax.fori_loop(
            0, N_CHUNKS, chunk_body,
            tuple(jnp.zeros((L,), jnp.float32) for _ in range(D // L)))
        total = accs[0]
        for j in range(1, D // L):
            total = total + accs[j]
        accv[...] = total
        pltpu.sync_copy(accv, out.at[wid])

    return sc_kernel


_sc_score = _make_sc_kernel()


def kernel(heads, rels, tails, entity_re, entity_im, r_re, r_im):
    parts = _sc_score(
        heads.astype(jnp.int32),
        rels.astype(jnp.int32),
        tails.astype(jnp.int32),
        entity_re.reshape(NUM_ENT // 8, 8, D),
        entity_im.reshape(NUM_ENT // 8, 8, D),
        r_re.reshape(NUM_REL // 2, 2 * D),
        r_im.reshape(NUM_REL // 2, 2 * D),
    )
    return jnp.sum(parts)


# speedup vs baseline: 1.0414x; 1.0414x over previous
import functools

import jax
import jax.numpy as jnp
from jax import lax
from jax.experimental import pallas as pl
from jax.experimental.pallas import tpu as pltpu
from jax.experimental.pallas import tpu_sc as plsc

D = 64
B = 16384
L = 16
NC = 2
NS = 16
NW = NC * NS
PER_W = B // NW
CHUNK = 128
NG = CHUNK // L
N_CHUNKS = PER_W // CHUNK
NUM_ENT = 1000000
NUM_REL = 1000


def _make_sc_kernel():
    mesh = plsc.VectorSubcoreMesh(core_axis_name="c", subcore_axis_name="s")

    @functools.partial(
        pl.kernel,
        out_type=jax.ShapeDtypeStruct((NW, L), jnp.float32),
        mesh=mesh,
        compiler_params=pltpu.CompilerParams(needs_layout_passes=False),
        scratch_types=[
            pltpu.VMEM((CHUNK,), jnp.int32),
            pltpu.VMEM((CHUNK,), jnp.int32),
            pltpu.VMEM((CHUNK,), jnp.int32),
            pltpu.VMEM((CHUNK,), jnp.int32),
            pltpu.VMEM((CHUNK, 2 * D), jnp.float32),
            pltpu.VMEM((CHUNK, 2 * D), jnp.float32),
            pltpu.VMEM((L, 8, D), jnp.float32),
            pltpu.VMEM((L, 8, D), jnp.float32),
            pltpu.VMEM((L, 8, D), jnp.float32),
            pltpu.VMEM((L, 8, D), jnp.float32),
            pltpu.VMEM((L,), jnp.float32),
            pltpu.SemaphoreType.DMA,
        ],
    )
    def sc_kernel(heads, rels, tails, ere3, eim3, rre, rim, out,
                  hidx, ridx, tidx, rdr, rbre, rbim,
                  bhre, bhim, btre, btim, accv, sem):
        wid = lax.axis_index("s") * NC + lax.axis_index("c")
        base = wid * PER_W

        def chunk_body(ck, accs):
            off = base + ck * CHUNK
            pltpu.sync_copy(heads.at[pl.ds(off, CHUNK)], hidx)
            pltpu.sync_copy(rels.at[pl.ds(off, CHUNK)], ridx)
            pltpu.sync_copy(tails.at[pl.ds(off, CHUNK)], tidx)
            for g in range(NG):
                sl = pl.ds(g * L, L)
                rdr[sl] = lax.shift_right_logical(ridx[sl], 1)
            cr1 = pltpu.async_copy(rre.at[rdr], rbre, sem)
            cr2 = pltpu.async_copy(rim.at[rdr], rbim, sem)
            cr1.wait()
            cr2.wait()

            def group_body(g, a):
                sl = pl.ds(g * L, L)
                hv = hidx[sl]
                tv = tidx[sl]
                rv = ridx[sl]
                hblk = lax.shift_right_logical(hv, 3)
                tblk = lax.shift_right_logical(tv, 3)
                hsub = lax.bitwise_and(hv, 7)
                tsub = lax.bitwise_and(tv, 7)
                rhalf = lax.bitwise_and(rv, 1) * D

                copies = []
                for l in range(L):
                    copies.append(pltpu.async_copy(
                        ere3.at[hblk[l]], bhre.at[l], sem))
                    copies.append(pltpu.async_copy(
                        eim3.at[hblk[l]], bhim.at[l], sem))
                    copies.append(pltpu.async_copy(
                        ere3.at[tblk[l]], btre.at[l], sem))
                    copies.append(pltpu.async_copy(
                        eim3.at[tblk[l]], btim.at[l], sem))

                new = list(a)
                for l in range(L):
                    for c in copies[4 * l:4 * l + 4]:
                        c.wait()
                    t = g * L + l
                    hs = hsub[l]
                    ts = tsub[l]
                    rh = rhalf[l]
                    for j in range(D // L):
                        dsl = pl.ds(j * L, L)
                        vhr = bhre[l, hs, dsl]
                        vhi = bhim[l, hs, dsl]
                        vtr = btre[l, ts, dsl]
                        vti = btim[l, ts, dsl]
                        rsl = pl.ds(rh + j * L, L)
                        vrr = rbre[t, rsl]
                        vri = rbim[t, rsl]
                        new[j] = (new[j] + vrr * (vhr * vtr + vhi * vti)
                                  + vri * (vhr * vti - vhi * vtr))
                return tuple(new)

            return lax.fori_loop(0, NG, group_body, accs)

        accs = lax.fori_loop(
            0, N_CHUNKS, chunk_body,
            tuple(jnp.zeros((L,), jnp.float32) for _ in range(D // L)))
        total = accs[0]
        for j in range(1, D // L):
            total = total + accs[j]
        accv[...] = total
        pltpu.sync_copy(accv, out.at[wid])

    return sc_kernel


_sc_score = _make_sc_kernel()


def kernel(heads, rels, tails, entity_re, entity_im, r_re, r_im):
    parts = _sc_score(
        heads.astype(jnp.int32),
        rels.astype(jnp.int32),
        tails.astype(jnp.int32),
        entity_re.reshape(NUM_ENT // 8, 8, D),
        entity_im.reshape(NUM_ENT // 8, 8, D),
        r_re.reshape(NUM_REL // 2, 2 * D),
        r_im.reshape(NUM_REL // 2, 2 * D),
    )
    return jnp.sum(parts)


# speedup vs baseline: 1.0457x; 1.0041x over previous
import functools

import jax
import jax.numpy as jnp
from jax import lax
from jax.experimental import pallas as pl
from jax.experimental.pallas import tpu as pltpu
from jax.experimental.pallas import tpu_sc as plsc

D = 64
B = 16384
L = 16
NC = 2
NS = 16
NW = NC * NS
PER_W = B // NW
CHUNK = 128
NG = CHUNK // L
N_CHUNKS = PER_W // CHUNK
NUM_ENT = 1000000
NUM_REL = 1000


def _make_sc_kernel():
    mesh = plsc.VectorSubcoreMesh(core_axis_name="c", subcore_axis_name="s")

    @functools.partial(
        pl.kernel,
        out_type=jax.ShapeDtypeStruct((NW, L), jnp.float32),
        mesh=mesh,
        compiler_params=pltpu.CompilerParams(needs_layout_passes=False),
        scratch_types=[
            pltpu.VMEM((CHUNK,), jnp.int32),
            pltpu.VMEM((CHUNK,), jnp.int32),
            pltpu.VMEM((CHUNK,), jnp.int32),
            pltpu.VMEM((CHUNK,), jnp.int32),
            pltpu.VMEM((CHUNK, 2 * D), jnp.float32),
            pltpu.VMEM((CHUNK, 2 * D), jnp.float32),
            pltpu.VMEM((L, 8, D), jnp.float32),
            pltpu.VMEM((L, 8, D), jnp.float32),
            pltpu.VMEM((L, 8, D), jnp.float32),
            pltpu.VMEM((L, 8, D), jnp.float32),
            pltpu.VMEM((L,), jnp.float32),
            pltpu.SemaphoreType.DMA,
        ],
    )
    def sc_kernel(heads, rels, tails, ere3, eim3, rre, rim, out,
                  hidx, ridx, tidx, rdr, rbre, rbim,
                  bhre, bhim, btre, btim, accv, sem):
        wid = lax.axis_index("s") * NC + lax.axis_index("c")
        base = wid * PER_W

        def chunk_body(ck, accs):
            off = base + ck * CHUNK
            ci_r = pltpu.async_copy(rels.at[pl.ds(off, CHUNK)], ridx, sem)
            ci_h = pltpu.async_copy(heads.at[pl.ds(off, CHUNK)], hidx, sem)
            ci_t = pltpu.async_copy(tails.at[pl.ds(off, CHUNK)], tidx, sem)
            ci_r.wait()
            ci_h.wait()
            ci_t.wait()
            for g in range(NG):
                sl = pl.ds(g * L, L)
                rdr[sl] = lax.shift_right_logical(ridx[sl], 1)
            cr1 = pltpu.async_copy(rre.at[rdr], rbre, sem)
            cr2 = pltpu.async_copy(rim.at[rdr], rbim, sem)
            cr1.wait()
            cr2.wait()

            def group_body(g, a):
                sl = pl.ds(g * L, L)
                hv = hidx[sl]
                tv = tidx[sl]
                rv = ridx[sl]
                hblk = lax.shift_right_logical(hv, 3)
                tblk = lax.shift_right_logical(tv, 3)
                hsub = lax.bitwise_and(hv, 7)
                tsub = lax.bitwise_and(tv, 7)
                rhalf = lax.bitwise_and(rv, 1) * D

                copies = []
                for l in range(L):
                    copies.append(pltpu.async_copy(
                        ere3.at[hblk[l]], bhre.at[l], sem))
                    copies.append(pltpu.async_copy(
                        eim3.at[hblk[l]], bhim.at[l], sem))
                    copies.append(pltpu.async_copy(
                        ere3.at[tblk[l]], btre.at[l], sem))
                    copies.append(pltpu.async_copy(
                        eim3.at[tblk[l]], btim.at[l], sem))

                new = list(a)
                for l in range(L):
                    for c in copies[4 * l:4 * l + 4]:
                        c.wait()
                    t = g * L + l
                    hs = hsub[l]
                    ts = tsub[l]
                    rh = rhalf[l]
                    for j in range(D // L):
                        dsl = pl.ds(j * L, L)
                        vhr = bhre[l, hs, dsl]
                        vhi = bhim[l, hs, dsl]
                        vtr = btre[l, ts, dsl]
                        vti = btim[l, ts, dsl]
                        rsl = pl.ds(rh + j * L, L)
                        vrr = rbre[t, rsl]
                        vri = rbim[t, rsl]
                        new[j] = (new[j] + vrr * (vhr * vtr + vhi * vti)
                                  + vri * (vhr * vti - vhi * vtr))
                return tuple(new)

            return lax.fori_loop(0, NG, group_body, accs)

        accs = lax.fori_loop(
            0, N_CHUNKS, chunk_body,
            tuple(jnp.zeros((L,), jnp.float32) for _ in range(D // L)))
        total = accs[0]
        for j in range(1, D // L):
            total = total + accs[j]
        accv[...] = total
        pltpu.sync_copy(accv, out.at[wid])

    return sc_kernel


_sc_score = _make_sc_kernel()


def kernel(heads, rels, tails, entity_re, entity_im, r_re, r_im):
    parts = _sc_score(
        heads.astype(jnp.int32),
        rels.astype(jnp.int32),
        tails.astype(jnp.int32),
        entity_re.reshape(NUM_ENT // 8, 8, D),
        entity_im.reshape(NUM_ENT // 8, 8, D),
        r_re.reshape(NUM_REL // 2, 2 * D),
        r_im.reshape(NUM_REL // 2, 2 * D),
    )
    return jnp.sum(parts)
